# Initial kernel scaffold; baseline (speedup 1.0000x reference)
import jax
import jax.numpy as jnp
from jax.experimental import pallas as pl


def kernel(z_e, W):
    raise NotImplementedError("write your pallas kernel here")


# speedup vs baseline: 1.4953x; 1.4953x over previous
import jax
import jax.numpy as jnp
from jax import lax
from jax.experimental import pallas as pl

NUM_EMBEDDINGS = 1024
EMBEDDING_DIM = 32
COMMITMENT_COST = 0.25
TOPO_WEIGHT = 0.35
GRID_SIZE = 32
BATCH = 64
H = 32
W_SP = 32
M_BLK = H * W_SP
N_TOTAL = BATCH * M_BLK * EMBEDDING_DIM


def _vq_kernel(z_ref, w_ref, zq_ref, idx_ref, vq_ref, topo_ref):
    pid = pl.program_id(0)

    w = w_ref[...]
    w_norm2 = jnp.sum(w * w, axis=1, keepdims=True)
    w_n = w / jnp.maximum(jnp.sqrt(w_norm2), 1e-12)

    zb = z_ref[0]
    z = jnp.transpose(zb, (1, 2, 0)).reshape(M_BLK, EMBEDDING_DIM)
    z_norm2 = jnp.sum(z * z, axis=1, keepdims=True)
    z_n = z / jnp.maximum(jnp.sqrt(z_norm2), 1e-12)

    scores = jnp.dot(z_n, w_n.T, preferred_element_type=jnp.float32)
    d = 2.0 - 2.0 * scores
    d_min = jnp.min(d, axis=1, keepdims=True)
    cols = lax.broadcasted_iota(jnp.int32, (M_BLK, NUM_EMBEDDINGS), 1)
    idx = jnp.min(jnp.where(d == d_min, cols, NUM_EMBEDDINGS), axis=1)

    onehot = (cols == idx[:, None]).astype(jnp.float32)
    z_q = jnp.dot(onehot, w_n, preferred_element_type=jnp.float32)

    idx_ref[0] = idx.reshape(H, W_SP)
    zq_ref[0] = jnp.transpose(z_q.reshape(H, W_SP, EMBEDDING_DIM), (2, 0, 1))

    @pl.when(pid == 0)
    def _init():
        vq_ref[...] = jnp.zeros((1, 1), jnp.float32)
        g = w_n.reshape(GRID_SIZE, GRID_SIZE, EMBEDDING_DIM)
        dh = g[:, 1:, :] - g[:, :-1, :]
        dv = g[1:, :, :] - g[:-1, :, :]
        wh = g[:, 0, :] - g[:, -1, :]
        wv = g[0, :, :] - g[-1, :, :]
        t = (jnp.sum(dh * dh) / dh.size + jnp.sum(dv * dv) / dv.size
             + jnp.sum(wh * wh) / wh.size + jnp.sum(wv * wv) / wv.size)
        topo_ref[...] = (TOPO_WEIGHT * t).reshape(1, 1)

    diff = z_q - z_n
    sq = jnp.sum(diff * diff)
    vq_ref[...] += (sq * ((1.0 + COMMITMENT_COST) / N_TOTAL)).reshape(1, 1)


def kernel(z_e, W):
    zq, idx, vq, topo = pl.pallas_call(
        _vq_kernel,
        grid=(BATCH,),
        in_specs=[
            pl.BlockSpec((1, EMBEDDING_DIM, H, W_SP), lambda i: (i, 0, 0, 0)),
            pl.BlockSpec((NUM_EMBEDDINGS, EMBEDDING_DIM), lambda i: (0, 0)),
        ],
        out_specs=[
            pl.BlockSpec((1, EMBEDDING_DIM, H, W_SP), lambda i: (i, 0, 0, 0)),
            pl.BlockSpec((1, H, W_SP), lambda i: (i, 0, 0)),
            pl.BlockSpec((1, 1), lambda i: (0, 0)),
            pl.BlockSpec((1, 1), lambda i: (0, 0)),
        ],
        out_shape=[
            jax.ShapeDtypeStruct((BATCH, EMBEDDING_DIM, H, W_SP), jnp.float32),
            jax.ShapeDtypeStruct((BATCH, H, W_SP), jnp.int32),
            jax.ShapeDtypeStruct((1, 1), jnp.float32),
            jax.ShapeDtypeStruct((1, 1), jnp.float32),
        ],
    )(z_e, W)
    return (zq, vq.reshape(()), idx, topo.reshape(()))


# speedup vs baseline: 2.8224x; 1.8875x over previous
import jax
import jax.numpy as jnp
from jax import lax
from jax.experimental import pallas as pl

NUM_EMBEDDINGS = 1024
EMBEDDING_DIM = 32
COMMITMENT_COST = 0.25
TOPO_WEIGHT = 0.35
GRID_SIZE = 32
BATCH = 64
H = 32
W_SP = 32
PIX = H * W_SP
N_TOTAL = BATCH * PIX * EMBEDDING_DIM


def _vq_kernel(z_ref, w_ref, zq_ref, idx_ref, vq_ref, topo_ref):
    pid = pl.program_id(0)

    w = w_ref[...]
    w_norm2 = jnp.sum(w * w, axis=1, keepdims=True)
    w_n = w / jnp.maximum(jnp.sqrt(w_norm2), 1e-12)

    z = z_ref[0]
    z_norm2 = jnp.sum(z * z, axis=0, keepdims=True)
    z_n = z / jnp.maximum(jnp.sqrt(z_norm2), 1e-12)

    scores = jnp.dot(w_n, z_n, preferred_element_type=jnp.float32)
    d = 2.0 - 2.0 * scores
    d_min = jnp.min(d, axis=0, keepdims=True)
    rows = lax.broadcasted_iota(jnp.int32, (NUM_EMBEDDINGS, PIX), 0)
    idx = jnp.min(jnp.where(d == d_min, rows, NUM_EMBEDDINGS), axis=0,
                  keepdims=True)

    onehot = (rows == idx).astype(jnp.float32)
    z_q = lax.dot_general(w_n, onehot, (((0,), (0,)), ((), ())),
                          preferred_element_type=jnp.float32)

    idx_ref[0] = idx
    zq_ref[0] = z_q

    @pl.when(pid == 0)
    def _init():
        vq_ref[...] = jnp.zeros((1, 1), jnp.float32)
        g = w_n.reshape(GRID_SIZE, GRID_SIZE, EMBEDDING_DIM)
        dh = g[:, 1:, :] - g[:, :-1, :]
        dv = g[1:, :, :] - g[:-1, :, :]
        wh = g[:, 0, :] - g[:, -1, :]
        wv = g[0, :, :] - g[-1, :, :]
        t = (jnp.sum(dh * dh) / dh.size + jnp.sum(dv * dv) / dv.size
             + jnp.sum(wh * wh) / wh.size + jnp.sum(wv * wv) / wv.size)
        topo_ref[...] = (TOPO_WEIGHT * t).reshape(1, 1)

    sq = jnp.sum(d_min)
    vq_ref[...] += (sq * ((1.0 + COMMITMENT_COST) / N_TOTAL)).reshape(1, 1)


def kernel(z_e, W):
    z_flat = z_e.reshape(BATCH, EMBEDDING_DIM, PIX)
    zq, idx, vq, topo = pl.pallas_call(
        _vq_kernel,
        grid=(BATCH,),
        in_specs=[
            pl.BlockSpec((1, EMBEDDING_DIM, PIX), lambda i: (i, 0, 0)),
            pl.BlockSpec((NUM_EMBEDDINGS, EMBEDDING_DIM), lambda i: (0, 0)),
        ],
        out_specs=[
            pl.BlockSpec((1, EMBEDDING_DIM, PIX), lambda i: (i, 0, 0)),
            pl.BlockSpec((1, 1, PIX), lambda i: (i, 0, 0)),
            pl.BlockSpec((1, 1), lambda i: (0, 0)),
            pl.BlockSpec((1, 1), lambda i: (0, 0)),
        ],
        out_shape=[
            jax.ShapeDtypeStruct((BATCH, EMBEDDING_DIM, PIX), jnp.float32),
            jax.ShapeDtypeStruct((BATCH, 1, PIX), jnp.int32),
            jax.ShapeDtypeStruct((1, 1), jnp.float32),
            jax.ShapeDtypeStruct((1, 1), jnp.float32),
        ],
    )(z_flat, W)
    return (zq.reshape(BATCH, EMBEDDING_DIM, H, W_SP), vq.reshape(()),
            idx.reshape(BATCH, H, W_SP), topo.reshape(()))
